# baseline (speedup 1.0000x reference)
import jax
import jax.numpy as jnp
from jax import lax
from jax.experimental import pallas as pl
from jax.experimental.pallas import tpu as pltpu

_ROWS_PER_BLOCK = 512

_DN_T = (((1,), (1,)), ((), ()))


def _gcn_allpass_kernel(y0_ref, a_ref, rho_ref, out_ref, a16, ya, yb):
    nblk = pl.num_programs(0)
    i = pl.program_id(0)
    rb = a_ref.shape[0]
    dg = out_ref.shape[0]
    nb = dg // (y0_ref.shape[0] - dg)

    ab = a_ref[...].astype(jnp.bfloat16)
    a16[pl.ds(i * rb, rb), :] = ab
    y = (y0_ref[...] + rho_ref[:, 0:1]).astype(jnp.bfloat16)
    ya[:, pl.ds(i * rb, rb)] = lax.dot_general(
        y, ab, _DN_T, preferred_element_type=jnp.float32)

    @pl.when(i == nblk - 1)
    def _tail():
        def one_pass(src, dst, p):
            yp = (src[...] + rho_ref[:, p:p + 1]).astype(jnp.bfloat16)
            for j in range(nblk):
                dst[:, j * rb:(j + 1) * rb] = lax.dot_general(
                    yp, a16[j * rb:(j + 1) * rb, :], _DN_T,
                    preferred_element_type=jnp.float32)

        one_pass(ya, yb, 1)
        one_pass(yb, ya, 2)

        yp = (ya[...] + rho_ref[:, 3:4]).astype(jnp.bfloat16)
        for j in range(nblk):
            res = lax.dot_general(yp, a16[j * rb:(j + 1) * rb, :], _DN_T,
                                  preferred_element_type=jnp.float32)
            comb = res[:dg, :] + jnp.concatenate([res[dg:, :]] * nb, axis=0)
            out_ref[:, j * rb:(j + 1) * rb] = comb


def kernel(x, adj, W1, b1, W2, b2, W3, b3, W4, b4):
    nb, in_dim, n = x.shape
    out_dim = W4.shape[0]
    f32 = jnp.float32

    c4 = W4.T
    c3 = W3.T @ c4
    c2 = W2.T @ c3
    c1 = W1.T @ c2

    w = nb * out_dim + out_dim
    rho = jnp.zeros((w, 128), f32)
    rho = rho.at[nb * out_dim:, 0].set(b1 @ c2)
    rho = rho.at[nb * out_dim:, 1].set(b2 @ c3)
    rho = rho.at[nb * out_dim:, 2].set(b3 @ c4)
    rho = rho.at[nb * out_dim:, 3].set(b4)

    g0 = jnp.einsum('bcn,cd->bdn', x, c1).reshape(nb * out_dim, n)
    y0 = jnp.concatenate([g0, jnp.zeros((out_dim, n), f32)], axis=0)

    rb = _ROWS_PER_BLOCK
    nblk = n // rb

    out = pl.pallas_call(
        _gcn_allpass_kernel,
        grid=(nblk,),
        in_specs=[
            pl.BlockSpec((w, n), lambda i: (0, 0)),
            pl.BlockSpec((rb, n), lambda i: (i, 0)),
            pl.BlockSpec((w, 128), lambda i: (0, 0)),
        ],
        out_specs=pl.BlockSpec((nb * out_dim, n), lambda i: (0, 0)),
        out_shape=jax.ShapeDtypeStruct((nb * out_dim, n), f32),
        scratch_shapes=[
            pltpu.VMEM((n, n), jnp.bfloat16),
            pltpu.VMEM((w, n), f32),
            pltpu.VMEM((w, n), f32),
        ],
        compiler_params=pltpu.CompilerParams(
            vmem_limit_bytes=100 * 1024 * 1024,
        ),
    )(y0, adj, rho)

    return out.reshape(nb, out_dim, n)


# speedup vs baseline: 1.0379x; 1.0379x over previous
import jax
import jax.numpy as jnp
from jax.experimental import pallas as pl
from jax.experimental.pallas import tpu as pltpu

_STREAM_BLOCK = 256
_TAIL_BLOCK = 512


def _gcn_allpass_kernel(m0_ref, a_up_ref, a_lo_ref, rho_ref, out_ref,
                        a16, ma, mb, m16):
    nstep = pl.num_programs(0)
    i = pl.program_id(0)
    rb = a_up_ref.shape[0]
    n = a_up_ref.shape[1]
    half = n // 2
    dg = out_ref.shape[0]
    nb = dg // (m0_ref.shape[1] - dg)

    @pl.when(i == 0)
    def _prep():
        m16[...] = (m0_ref[...] + rho_ref[0:1, :]).astype(jnp.bfloat16)

    m = m16[...]
    au = a_up_ref[...].astype(jnp.bfloat16)
    al = a_lo_ref[...].astype(jnp.bfloat16)
    a16[pl.ds(i * rb, rb), :] = au
    a16[pl.ds(half + i * rb, rb), :] = al
    ma[pl.ds(i * rb, rb), :] = jnp.dot(au, m, preferred_element_type=jnp.float32)
    ma[pl.ds(half + i * rb, rb), :] = jnp.dot(
        al, m, preferred_element_type=jnp.float32)

    @pl.when(i == nstep - 1)
    def _tail():
        tb = _TAIL_BLOCK
        nchunk = n // tb

        def one_pass(src, dst, p):
            mp = (src[...] + rho_ref[p:p + 1, :]).astype(jnp.bfloat16)
            for j in range(nchunk):
                dst[j * tb:(j + 1) * tb, :] = jnp.dot(
                    a16[j * tb:(j + 1) * tb, :], mp,
                    preferred_element_type=jnp.float32)

        one_pass(ma, mb, 1)
        one_pass(mb, ma, 2)

        mp = (ma[...] + rho_ref[3:4, :]).astype(jnp.bfloat16)
        for j in range(nchunk):
            res = jnp.dot(a16[j * tb:(j + 1) * tb, :], mp,
                          preferred_element_type=jnp.float32)
            comb = res[:, :dg] + jnp.concatenate([res[:, dg:]] * nb, axis=1)
            out_ref[:, j * tb:(j + 1) * tb] = comb.T


def kernel(x, adj, W1, b1, W2, b2, W3, b3, W4, b4):
    nb, in_dim, n = x.shape
    out_dim = W4.shape[0]
    f32 = jnp.float32

    c4 = W4.T
    c3 = W3.T @ c4
    c2 = W2.T @ c3
    c1 = W1.T @ c2

    w = nb * out_dim + out_dim
    rho = jnp.zeros((8, w), f32)
    rho = rho.at[0, nb * out_dim:].set(b1 @ c2)
    rho = rho.at[1, nb * out_dim:].set(b2 @ c3)
    rho = rho.at[2, nb * out_dim:].set(b3 @ c4)
    rho = rho.at[3, nb * out_dim:].set(b4)

    h0 = jnp.transpose(x, (2, 0, 1))
    g0 = jnp.einsum('nbc,cd->nbd', h0, c1).reshape(n, nb * out_dim)
    m0 = jnp.concatenate([g0, jnp.zeros((n, out_dim), f32)], axis=1)

    rb = _STREAM_BLOCK
    nstep = (n // 2) // rb
    half_blk = nstep

    out = pl.pallas_call(
        _gcn_allpass_kernel,
        grid=(nstep,),
        in_specs=[
            pl.BlockSpec((n, w), lambda i: (0, 0)),
            pl.BlockSpec((rb, n), lambda i: (i, 0)),
            pl.BlockSpec((rb, n), lambda i, h=half_blk: (i + h, 0)),
            pl.BlockSpec((8, w), lambda i: (0, 0)),
        ],
        out_specs=pl.BlockSpec((nb * out_dim, n), lambda i: (0, 0)),
        out_shape=jax.ShapeDtypeStruct((nb * out_dim, n), f32),
        scratch_shapes=[
            pltpu.VMEM((n, n), jnp.bfloat16),
            pltpu.VMEM((n, w), f32),
            pltpu.VMEM((n, w), f32),
            pltpu.VMEM((n, w), jnp.bfloat16),
        ],
        compiler_params=pltpu.CompilerParams(
            vmem_limit_bytes=100 * 1024 * 1024,
        ),
    )(m0, adj, adj, rho)

    return out.reshape(nb, out_dim, n)


# speedup vs baseline: 1.1198x; 1.0789x over previous
import jax
import jax.numpy as jnp
from jax.experimental import pallas as pl
from jax.experimental.pallas import tpu as pltpu

_STREAM_BLOCK = 512
_TAIL_BLOCK = 512


def _gcn_allpass_kernel(m0_ref, a_ref, rho_ref, out_ref, a16, ma, mb):
    nstep = pl.num_programs(0)
    i = pl.program_id(0)
    rb = a_ref.shape[0]
    n = a_ref.shape[1]
    dg = out_ref.shape[0]
    nb = dg // (m0_ref.shape[1] - dg)

    a = a_ref[...]
    a16[pl.ds(i * rb, rb), :] = a.astype(jnp.bfloat16)
    ma[pl.ds(i * rb, rb), :] = jnp.dot(a, m0_ref[...],
                                       preferred_element_type=jnp.float32)

    @pl.when(i == nstep - 1)
    def _tail():
        tb = _TAIL_BLOCK
        nchunk = n // tb

        def one_pass(src, dst, p):
            mp = (src[...] + rho_ref[p:p + 1, :]).astype(jnp.bfloat16)
            for j in range(nchunk):
                dst[j * tb:(j + 1) * tb, :] = jnp.dot(
                    a16[j * tb:(j + 1) * tb, :], mp,
                    preferred_element_type=jnp.float32)

        one_pass(ma, mb, 1)
        one_pass(mb, ma, 2)

        mp = (ma[...] + rho_ref[3:4, :]).astype(jnp.bfloat16)
        for j in range(nchunk):
            res = jnp.dot(a16[j * tb:(j + 1) * tb, :], mp,
                          preferred_element_type=jnp.float32)
            comb = res[:, :dg] + jnp.concatenate([res[:, dg:]] * nb, axis=1)
            out_ref[:, j * tb:(j + 1) * tb] = comb.T


def kernel(x, adj, W1, b1, W2, b2, W3, b3, W4, b4):
    nb, in_dim, n = x.shape
    out_dim = W4.shape[0]
    f32 = jnp.float32

    c4 = W4.T
    c3 = W3.T @ c4
    c2 = W2.T @ c3
    c1 = W1.T @ c2

    w = nb * out_dim + out_dim
    rho = jnp.zeros((8, w), f32)
    rho = rho.at[1, nb * out_dim:].set(b2 @ c3)
    rho = rho.at[2, nb * out_dim:].set(b3 @ c4)
    rho = rho.at[3, nb * out_dim:].set(b4)

    h0 = jnp.transpose(x, (2, 0, 1))
    g0 = jnp.einsum('nbc,cd->nbd', h0, c1).reshape(n, nb * out_dim)
    p0 = jnp.broadcast_to(b1 @ c2, (n, out_dim))
    m0 = jnp.concatenate([g0, p0], axis=1)

    rb = _STREAM_BLOCK

    out = pl.pallas_call(
        _gcn_allpass_kernel,
        grid=(n // rb,),
        in_specs=[
            pl.BlockSpec((n, w), lambda i: (0, 0)),
            pl.BlockSpec((rb, n), lambda i: (i, 0)),
            pl.BlockSpec((8, w), lambda i: (0, 0)),
        ],
        out_specs=pl.BlockSpec((nb * out_dim, n), lambda i: (0, 0)),
        out_shape=jax.ShapeDtypeStruct((nb * out_dim, n), f32),
        scratch_shapes=[
            pltpu.VMEM((n, n), jnp.bfloat16),
            pltpu.VMEM((n, w), f32),
            pltpu.VMEM((n, w), f32),
        ],
        compiler_params=pltpu.CompilerParams(
            vmem_limit_bytes=100 * 1024 * 1024,
        ),
    )(m0, adj, rho)

    return out.reshape(nb, out_dim, n)
